# baseline (speedup 1.0000x reference)
import functools

import jax
import jax.numpy as jnp
from jax import lax
from jax.experimental import pallas as pl
from jax.experimental.pallas import tpu as pltpu
from jax.experimental.pallas import tpu_sc as plsc

_N, _D, _H, _E, _K = 2048, 768, 768, 8, 2
_BLK = 256
_NBLK = _N * _K // _BLK + _E
_CAP = _NBLK * _BLK
_CHUNK = 256

_NW = 32
_TPW = _N // _NW
_LANES = 16


def _router_body(x_ref, wg_ref, bg_ref, d0_ref, d1_ref, w0_ref, w1_ref,
                 blk_ref, cbt, cbuf):
    x = x_ref[...]
    scores = jnp.dot(x, wg_ref[...],
                     preferred_element_type=jnp.float32) + bg_ref[...]
    eidx = jax.lax.broadcasted_iota(jnp.int32, scores.shape, 1)
    m1 = jnp.max(scores, axis=1, keepdims=True)
    i1 = jnp.min(jnp.where(scores == m1, eidx, _E), axis=1, keepdims=True)
    oh1 = eidx == i1
    neg = jnp.where(oh1, -jnp.inf, scores)
    m2 = jnp.max(neg, axis=1, keepdims=True)
    i2 = jnp.min(jnp.where(neg == m2, eidx, _E), axis=1, keepdims=True)
    oh2 = eidx == i2
    ex = jnp.exp(scores - m1)
    p = ex / jnp.sum(ex, axis=1, keepdims=True)
    wm = p * (oh1 | oh2).astype(jnp.float32)
    w = wm / (jnp.sum(wm, axis=1, keepdims=True) + 1e-8)
    w0_ref[...] = jnp.broadcast_to(
        jnp.sum(jnp.where(oh1, w, 0.0), axis=1, keepdims=True), (_N, _LANES))
    w1_ref[...] = jnp.broadcast_to(
        jnp.sum(jnp.where(oh2, w, 0.0), axis=1, keepdims=True), (_N, _LANES))

    cbuf[...] = oh1.astype(jnp.float32) + oh2.astype(jnp.float32)
    ti = jax.lax.broadcasted_iota(jnp.int32, (_CHUNK, _CHUNK), 0)
    tj = jax.lax.broadcasted_iota(jnp.int32, (_CHUNK, _CHUNK), 1)
    tstrict = (tj < ti).astype(jnp.float32)

    def step(k, carry):
        rows = pl.ds(k * _CHUNK, _CHUNK)
        chunk = cbuf[rows, :]
        cbt[rows, :] = carry + jnp.dot(
            tstrict, chunk, preferred_element_type=jnp.float32,
            precision=jax.lax.Precision.HIGHEST)
        return carry + jnp.sum(chunk, axis=0, keepdims=True)

    counts = lax.fori_loop(0, _N // _CHUNK, step,
                           jnp.zeros((1, _E), jnp.float32))
    pc = jnp.ceil(counts / _BLK) * _BLK
    ui = jax.lax.broadcasted_iota(jnp.int32, (_E, _E), 0)
    uj = jax.lax.broadcasted_iota(jnp.int32, (_E, _E), 1)
    ustrict = (ui < uj).astype(jnp.float32)
    po = jnp.dot(pc, ustrict, preferred_element_type=jnp.float32,
                 precision=jax.lax.Precision.HIGHEST)

    base = cbt[...] + po
    d0_ref[...] = jnp.sum(jnp.where(oh1, base, 0.0), axis=1,
                          keepdims=True).astype(jnp.int32)
    d1_ref[...] = jnp.sum(jnp.where(oh2, base, 0.0), axis=1,
                          keepdims=True).astype(jnp.int32)

    po_next = po + pc
    bpos = jax.lax.broadcasted_iota(
        jnp.int32, (_NBLK, _E), 0).astype(jnp.float32) * float(_BLK)
    blk = jnp.sum((bpos >= po_next).astype(jnp.int32), axis=1, keepdims=True)
    blk_ref[...] = jnp.minimum(blk, _E - 1)


def _router(x, Wg, bg):
    return pl.pallas_call(
        _router_body,
        out_shape=[
            jax.ShapeDtypeStruct((_N, 1), jnp.int32),
            jax.ShapeDtypeStruct((_N, 1), jnp.int32),
            jax.ShapeDtypeStruct((_N, _LANES), jnp.float32),
            jax.ShapeDtypeStruct((_N, _LANES), jnp.float32),
            jax.ShapeDtypeStruct((_NBLK, 1), jnp.int32),
        ],
        scratch_shapes=[pltpu.VMEM((_N, _E), jnp.float32),
                        pltpu.VMEM((_N, _E), jnp.float32)],
    )(x, Wg, bg.reshape(1, _E))


def _dispatch_body(x_hbm, d0_hbm, d1_hbm, xg_hbm, xv, d0v, d1v, sem):
    wid = lax.axis_index("s") * 2 + lax.axis_index("c")
    base = wid * _TPW
    pltpu.sync_copy(x_hbm.at[pl.ds(base, _TPW)], xv)
    pltpu.sync_copy(d0_hbm.at[pl.ds(base, _TPW)], d0v)
    pltpu.sync_copy(d1_hbm.at[pl.ds(base, _TPW)], d1v)
    cp0 = pltpu.async_copy(xv, xg_hbm.at[d0v], sem)
    cp1 = pltpu.async_copy(xv, xg_hbm.at[d1v], sem)
    cp0.wait()
    cp1.wait()


@functools.cache
def _dispatch():
    return pl.kernel(
        _dispatch_body,
        mesh=plsc.VectorSubcoreMesh(core_axis_name="c", subcore_axis_name="s"),
        out_type=jax.ShapeDtypeStruct((_CAP, _D), jnp.float32),
        scratch_types=[
            pltpu.VMEM((_TPW, _D), jnp.float32),
            pltpu.VMEM((_TPW,), jnp.int32),
            pltpu.VMEM((_TPW,), jnp.int32),
            pltpu.SemaphoreType.DMA,
        ],
    )


def _expert_body(blk_ref, xg_ref, w1_ref, b1_ref, w2_ref, b2_ref, y_ref):
    h = jnp.maximum(
        jnp.dot(xg_ref[...], w1_ref[0],
                preferred_element_type=jnp.float32) + b1_ref[0],
        0.0)
    y_ref[...] = jnp.dot(h, w2_ref[0],
                         preferred_element_type=jnp.float32) + b2_ref[0]


def _expert_mm(blk, xg, W1, b1, W2, b2):
    grid_spec = pltpu.PrefetchScalarGridSpec(
        num_scalar_prefetch=1,
        grid=(_NBLK,),
        in_specs=[
            pl.BlockSpec((_BLK, _D), lambda b, blk: (b, 0)),
            pl.BlockSpec((1, _D, _H), lambda b, blk: (blk[b], 0, 0)),
            pl.BlockSpec((1, 1, _H), lambda b, blk: (blk[b], 0, 0)),
            pl.BlockSpec((1, _H, _D), lambda b, blk: (blk[b], 0, 0)),
            pl.BlockSpec((1, 1, _D), lambda b, blk: (blk[b], 0, 0)),
        ],
        out_specs=pl.BlockSpec((_BLK, _D), lambda b, blk: (b, 0)),
    )
    return pl.pallas_call(
        _expert_body,
        grid_spec=grid_spec,
        out_shape=jax.ShapeDtypeStruct((_CAP, _D), jnp.float32),
        compiler_params=pltpu.CompilerParams(
            dimension_semantics=("arbitrary",)),
    )(blk, xg, W1, b1.reshape(_E, 1, _H), W2, b2.reshape(_E, 1, _D))


def _combine_body(y_hbm, d0_hbm, d1_hbm, w0_hbm, w1_hbm, out_hbm,
                  av, bv, d0v, d1v, w0v, w1v, sem):
    wid = lax.axis_index("s") * 2 + lax.axis_index("c")
    base = wid * _TPW
    pltpu.sync_copy(d0_hbm.at[pl.ds(base, _TPW)], d0v)
    pltpu.sync_copy(d1_hbm.at[pl.ds(base, _TPW)], d1v)
    pltpu.sync_copy(w0_hbm.at[pl.ds(base, _TPW)], w0v)
    pltpu.sync_copy(w1_hbm.at[pl.ds(base, _TPW)], w1v)
    cp0 = pltpu.async_copy(y_hbm.at[d0v], av, sem)
    cp1 = pltpu.async_copy(y_hbm.at[d1v], bv, sem)
    cp0.wait()
    cp1.wait()

    def tok(i, _):
        s0 = w0v[i]
        s1 = w1v[i]
        for j in range(_D // _LANES):
            cols = pl.ds(j * _LANES, _LANES)
            av[i, cols] = av[i, cols] * s0 + bv[i, cols] * s1
        return 0

    lax.fori_loop(0, _TPW, tok, 0)
    pltpu.sync_copy(av, out_hbm.at[pl.ds(base, _TPW)])


@functools.cache
def _combine():
    return pl.kernel(
        _combine_body,
        mesh=plsc.VectorSubcoreMesh(core_axis_name="c", subcore_axis_name="s"),
        out_type=jax.ShapeDtypeStruct((_N, _D), jnp.float32),
        scratch_types=[
            pltpu.VMEM((_TPW, _D), jnp.float32),
            pltpu.VMEM((_TPW, _D), jnp.float32),
            pltpu.VMEM((_TPW,), jnp.int32),
            pltpu.VMEM((_TPW,), jnp.int32),
            pltpu.VMEM((_TPW, _LANES), jnp.float32),
            pltpu.VMEM((_TPW, _LANES), jnp.float32),
            pltpu.SemaphoreType.DMA,
        ],
    )


def kernel(x, Wg, bg, W1, b1, W2, b2):
    d0, d1, w0, w1, blk = _router(x, Wg, bg)
    d0 = d0.reshape(_N)
    d1 = d1.reshape(_N)
    blk = blk.reshape(_NBLK)
    xg = _dispatch()(x, d0, d1)
    y = _expert_mm(blk, xg, W1.astype(jnp.bfloat16), b1,
                   W2.astype(jnp.bfloat16), b2)
    return _combine()(y, d0, d1, w0, w1)


# speedup vs baseline: 1.0405x; 1.0405x over previous
import functools

import jax
import jax.numpy as jnp
from jax import lax
from jax.experimental import pallas as pl
from jax.experimental.pallas import tpu as pltpu
from jax.experimental.pallas import tpu_sc as plsc

_N, _D, _H, _E, _K = 2048, 768, 768, 8, 2
_BLK = 128
_NBLK = _N * _K // _BLK + _E
_CAP = _NBLK * _BLK
_CHUNK = 256

_NW = 32
_TPW = _N // _NW
_LANES = 16


def _router_body(x_ref, wg_ref, bg_ref, d0_ref, d1_ref, w0_ref, w1_ref,
                 blk_ref, cbt, cbuf):
    x = x_ref[...]
    scores = jnp.dot(x, wg_ref[...],
                     preferred_element_type=jnp.float32) + bg_ref[...]
    eidx = jax.lax.broadcasted_iota(jnp.int32, scores.shape, 1)
    m1 = jnp.max(scores, axis=1, keepdims=True)
    i1 = jnp.min(jnp.where(scores == m1, eidx, _E), axis=1, keepdims=True)
    oh1 = eidx == i1
    neg = jnp.where(oh1, -jnp.inf, scores)
    m2 = jnp.max(neg, axis=1, keepdims=True)
    i2 = jnp.min(jnp.where(neg == m2, eidx, _E), axis=1, keepdims=True)
    oh2 = eidx == i2
    ex = jnp.exp(scores - m1)
    p = ex / jnp.sum(ex, axis=1, keepdims=True)
    wm = p * (oh1 | oh2).astype(jnp.float32)
    w = wm / (jnp.sum(wm, axis=1, keepdims=True) + 1e-8)
    w0_ref[...] = jnp.broadcast_to(
        jnp.sum(jnp.where(oh1, w, 0.0), axis=1, keepdims=True), (_N, _LANES))
    w1_ref[...] = jnp.broadcast_to(
        jnp.sum(jnp.where(oh2, w, 0.0), axis=1, keepdims=True), (_N, _LANES))

    cbuf[...] = oh1.astype(jnp.float32) + oh2.astype(jnp.float32)
    ti = jax.lax.broadcasted_iota(jnp.int32, (_CHUNK, _CHUNK), 0)
    tj = jax.lax.broadcasted_iota(jnp.int32, (_CHUNK, _CHUNK), 1)
    tstrict = (tj < ti).astype(jnp.float32)

    def step(k, carry):
        rows = pl.ds(k * _CHUNK, _CHUNK)
        chunk = cbuf[rows, :]
        cbt[rows, :] = carry + jnp.dot(
            tstrict, chunk, preferred_element_type=jnp.float32,
            precision=jax.lax.Precision.HIGHEST)
        return carry + jnp.sum(chunk, axis=0, keepdims=True)

    counts = lax.fori_loop(0, _N // _CHUNK, step,
                           jnp.zeros((1, _E), jnp.float32))
    pc = jnp.ceil(counts / _BLK) * _BLK
    ui = jax.lax.broadcasted_iota(jnp.int32, (_E, _E), 0)
    uj = jax.lax.broadcasted_iota(jnp.int32, (_E, _E), 1)
    ustrict = (ui < uj).astype(jnp.float32)
    po = jnp.dot(pc, ustrict, preferred_element_type=jnp.float32,
                 precision=jax.lax.Precision.HIGHEST)

    base = cbt[...] + po
    d0_ref[...] = jnp.sum(jnp.where(oh1, base, 0.0), axis=1,
                          keepdims=True).astype(jnp.int32)
    d1_ref[...] = jnp.sum(jnp.where(oh2, base, 0.0), axis=1,
                          keepdims=True).astype(jnp.int32)

    po_next = po + pc
    bpos = jax.lax.broadcasted_iota(
        jnp.int32, (_NBLK, _E), 0).astype(jnp.float32) * float(_BLK)
    blk = jnp.sum((bpos >= po_next).astype(jnp.int32), axis=1, keepdims=True)
    blk_ref[...] = jnp.minimum(blk, _E - 1)


def _router(x, Wg, bg):
    return pl.pallas_call(
        _router_body,
        out_shape=[
            jax.ShapeDtypeStruct((_N, 1), jnp.int32),
            jax.ShapeDtypeStruct((_N, 1), jnp.int32),
            jax.ShapeDtypeStruct((_N, _LANES), jnp.float32),
            jax.ShapeDtypeStruct((_N, _LANES), jnp.float32),
            jax.ShapeDtypeStruct((_NBLK, 1), jnp.int32),
        ],
        scratch_shapes=[pltpu.VMEM((_N, _E), jnp.float32),
                        pltpu.VMEM((_N, _E), jnp.float32)],
    )(x, Wg, bg.reshape(1, _E))


def _dispatch_body(x_hbm, d0_hbm, d1_hbm, xg_hbm, xv, d0v, d1v, sem):
    wid = lax.axis_index("s") * 2 + lax.axis_index("c")
    base = wid * _TPW
    pltpu.sync_copy(x_hbm.at[pl.ds(base, _TPW)], xv)
    pltpu.sync_copy(d0_hbm.at[pl.ds(base, _TPW)], d0v)
    pltpu.sync_copy(d1_hbm.at[pl.ds(base, _TPW)], d1v)
    cp0 = pltpu.async_copy(xv, xg_hbm.at[d0v], sem)
    cp1 = pltpu.async_copy(xv, xg_hbm.at[d1v], sem)
    cp0.wait()
    cp1.wait()


@functools.cache
def _dispatch():
    return pl.kernel(
        _dispatch_body,
        mesh=plsc.VectorSubcoreMesh(core_axis_name="c", subcore_axis_name="s"),
        out_type=jax.ShapeDtypeStruct((_CAP, _D), jnp.float32),
        scratch_types=[
            pltpu.VMEM((_TPW, _D), jnp.float32),
            pltpu.VMEM((_TPW,), jnp.int32),
            pltpu.VMEM((_TPW,), jnp.int32),
            pltpu.SemaphoreType.DMA,
        ],
    )


def _expert_body(blk_ref, xg_ref, w1_ref, b1_ref, w2_ref, b2_ref, y_ref):
    h = jnp.maximum(
        jnp.dot(xg_ref[...], w1_ref[0],
                preferred_element_type=jnp.float32) + b1_ref[0],
        0.0)
    y_ref[...] = jnp.dot(h, w2_ref[0],
                         preferred_element_type=jnp.float32) + b2_ref[0]


def _expert_mm(blk, xg, W1, b1, W2, b2):
    grid_spec = pltpu.PrefetchScalarGridSpec(
        num_scalar_prefetch=1,
        grid=(_NBLK,),
        in_specs=[
            pl.BlockSpec((_BLK, _D), lambda b, blk: (b, 0)),
            pl.BlockSpec((1, _D, _H), lambda b, blk: (blk[b], 0, 0)),
            pl.BlockSpec((1, 1, _H), lambda b, blk: (blk[b], 0, 0)),
            pl.BlockSpec((1, _H, _D), lambda b, blk: (blk[b], 0, 0)),
            pl.BlockSpec((1, 1, _D), lambda b, blk: (blk[b], 0, 0)),
        ],
        out_specs=pl.BlockSpec((_BLK, _D), lambda b, blk: (b, 0)),
    )
    return pl.pallas_call(
        _expert_body,
        grid_spec=grid_spec,
        out_shape=jax.ShapeDtypeStruct((_CAP, _D), jnp.float32),
        compiler_params=pltpu.CompilerParams(
            dimension_semantics=("arbitrary",)),
    )(blk, xg, W1, b1.reshape(_E, 1, _H), W2, b2.reshape(_E, 1, _D))


def _combine_body(y_hbm, d0_hbm, d1_hbm, w0_hbm, w1_hbm, out_hbm,
                  av, bv, d0v, d1v, w0v, w1v, sem):
    wid = lax.axis_index("s") * 2 + lax.axis_index("c")
    base = wid * _TPW
    pltpu.sync_copy(d0_hbm.at[pl.ds(base, _TPW)], d0v)
    pltpu.sync_copy(d1_hbm.at[pl.ds(base, _TPW)], d1v)
    pltpu.sync_copy(w0_hbm.at[pl.ds(base, _TPW)], w0v)
    pltpu.sync_copy(w1_hbm.at[pl.ds(base, _TPW)], w1v)
    cp0 = pltpu.async_copy(y_hbm.at[d0v], av, sem)
    cp1 = pltpu.async_copy(y_hbm.at[d1v], bv, sem)
    cp0.wait()
    cp1.wait()

    def tok(i, _):
        s0 = w0v[i]
        s1 = w1v[i]
        for j in range(_D // _LANES):
            cols = pl.ds(j * _LANES, _LANES)
            av[i, cols] = av[i, cols] * s0 + bv[i, cols] * s1
        return 0

    lax.fori_loop(0, _TPW, tok, 0)
    pltpu.sync_copy(av, out_hbm.at[pl.ds(base, _TPW)])


@functools.cache
def _combine():
    return pl.kernel(
        _combine_body,
        mesh=plsc.VectorSubcoreMesh(core_axis_name="c", subcore_axis_name="s"),
        out_type=jax.ShapeDtypeStruct((_N, _D), jnp.float32),
        scratch_types=[
            pltpu.VMEM((_TPW, _D), jnp.float32),
            pltpu.VMEM((_TPW, _D), jnp.float32),
            pltpu.VMEM((_TPW,), jnp.int32),
            pltpu.VMEM((_TPW,), jnp.int32),
            pltpu.VMEM((_TPW, _LANES), jnp.float32),
            pltpu.VMEM((_TPW, _LANES), jnp.float32),
            pltpu.SemaphoreType.DMA,
        ],
    )


def kernel(x, Wg, bg, W1, b1, W2, b2):
    d0, d1, w0, w1, blk = _router(x, Wg, bg)
    d0 = d0.reshape(_N)
    d1 = d1.reshape(_N)
    blk = blk.reshape(_NBLK)
    xg = _dispatch()(x, d0, d1)
    y = _expert_mm(blk, xg, W1, b1, W2, b2)
    return _combine()(y, d0, d1, w0, w1)


# speedup vs baseline: 1.7819x; 1.7126x over previous
import jax
import jax.numpy as jnp
from jax.experimental import pallas as pl
from jax.experimental.pallas import tpu as pltpu

_N, _D, _H, _E, _K = 2048, 768, 768, 8, 2


def _moe_body(x_ref, wg_ref, bg_ref, w1_ref, b1_ref, w2_ref, b2_ref,
              out_ref, gate):
    e = pl.program_id(0)

    @pl.when(e == 0)
    def _gate():
        x = x_ref[...]
        scores = jnp.dot(x, wg_ref[...],
                         preferred_element_type=jnp.float32) + bg_ref[...]
        eidx = jax.lax.broadcasted_iota(jnp.int32, scores.shape, 1)
        m1 = jnp.max(scores, axis=1, keepdims=True)
        i1 = jnp.min(jnp.where(scores == m1, eidx, _E), axis=1, keepdims=True)
        oh1 = eidx == i1
        neg = jnp.where(oh1, -jnp.inf, scores)
        m2 = jnp.max(neg, axis=1, keepdims=True)
        i2 = jnp.min(jnp.where(neg == m2, eidx, _E), axis=1, keepdims=True)
        oh2 = eidx == i2
        ex = jnp.exp(scores - m1)
        p = ex / jnp.sum(ex, axis=1, keepdims=True)
        wm = p * (oh1 | oh2).astype(jnp.float32)
        gate[...] = wm / (jnp.sum(wm, axis=1, keepdims=True) + 1e-8)

    x = x_ref[...]
    h = jnp.maximum(
        jnp.dot(x, w1_ref[0], preferred_element_type=jnp.float32) + b1_ref[0],
        0.0)
    o = jnp.dot(h, w2_ref[0], preferred_element_type=jnp.float32) + b2_ref[0]
    ge = gate[...]
    sel = (jax.lax.broadcasted_iota(jnp.int32, ge.shape, 1) == e)
    wcol = jnp.sum(jnp.where(sel, ge, 0.0), axis=1, keepdims=True)
    contrib = wcol * o

    @pl.when(e == 0)
    def _init():
        out_ref[...] = contrib

    @pl.when(e > 0)
    def _acc():
        out_ref[...] += contrib


def kernel(x, Wg, bg, W1, b1, W2, b2):
    return pl.pallas_call(
        _moe_body,
        grid=(_E,),
        in_specs=[
            pl.BlockSpec((_N, _D), lambda e: (0, 0)),
            pl.BlockSpec((_D, _E), lambda e: (0, 0)),
            pl.BlockSpec((1, _E), lambda e: (0, 0)),
            pl.BlockSpec((1, _D, _H), lambda e: (e, 0, 0)),
            pl.BlockSpec((1, 1, _H), lambda e: (e, 0, 0)),
            pl.BlockSpec((1, _H, _D), lambda e: (e, 0, 0)),
            pl.BlockSpec((1, 1, _D), lambda e: (e, 0, 0)),
        ],
        out_specs=pl.BlockSpec((_N, _D), lambda e: (0, 0)),
        out_shape=jax.ShapeDtypeStruct((_N, _D), jnp.float32),
        scratch_shapes=[
            pltpu.VMEM((_N, _E), jnp.float32),
        ],
        compiler_params=pltpu.CompilerParams(
            dimension_semantics=("arbitrary",)),
    )(x, Wg, bg.reshape(1, _E), W1, b1.reshape(_E, 1, _H),
      W2, b2.reshape(_E, 1, _D))
